# baseline (speedup 1.0000x reference)
import jax
import jax.numpy as jnp
from jax.experimental import pallas as pl


_ROWS = 512


def _mul_body(x_ref, m_ref, o_ref):
    o_ref[...] = x_ref[...] * m_ref[...][None, :, :]


def kernel(x, mask):
    B, S, D = x.shape
    m2 = mask.reshape(S, 1)
    grid = (B, S // _ROWS)
    out = pl.pallas_call(
        _mul_body,
        grid=grid,
        in_specs=[
            pl.BlockSpec((1, _ROWS, D), lambda b, r: (b, r, 0)),
            pl.BlockSpec((_ROWS, 1), lambda b, r: (r, 0)),
        ],
        out_specs=pl.BlockSpec((1, _ROWS, D), lambda b, r: (b, r, 0)),
        out_shape=jax.ShapeDtypeStruct((B, S, D), x.dtype),
    )(x, m2)
    return out
